# Initial kernel scaffold; baseline (speedup 1.0000x reference)
import jax
import jax.numpy as jnp
from jax import lax
from jax.experimental import pallas as pl
from jax.experimental.pallas import tpu as pltpu
from jax.experimental.pallas import tpu_sc as plsc

N = 10000
E = 320000
D_IN = 128
D_HID = 256
D_OUT = 64
HALF = D_HID // 2

NC = 2
NS = 16
RPT = N // NS
EB = 80

_MESH = dict(core_axis_name="c", subcore_axis_name="s")


def _edge_loop(nb, base, src_hbm, dst_hbm, g_hbm, slab, srcv, dstv, rows, sem):
    def body(i, carry):
        off = pl.multiple_of(base + i * EB, 8)
        pltpu.sync_copy(src_hbm.at[pl.ds(off, EB)], srcv)
        pltpu.sync_copy(dst_hbm.at[pl.ds(off, EB)], dstv)
        pltpu.async_copy(g_hbm.at[srcv], rows, sem).wait()
        pltpu.sync_copy(rows, slab.at[dstv], add=True)
        return carry
    lax.fori_loop(0, nb, body, 0)


def _deg_body(dst_hbm, ones_hbm, zrows_hbm, degA, degB, onesv, dstv, slab):
    c = lax.axis_index("c")
    s = lax.axis_index("s")
    r0 = s * RPT
    pltpu.sync_copy(zrows_hbm, slab.at[pl.ds(r0, RPT)])
    pltpu.sync_copy(ones_hbm, onesv)
    plsc.subcore_barrier()
    base = pl.multiple_of((c * NS + s) * (E // (NC * NS)), 8)
    nb = E // (NC * NS * EB)

    def body(i, carry):
        off = pl.multiple_of(base + i * EB, 8)
        pltpu.sync_copy(dst_hbm.at[pl.ds(off, EB)], dstv)
        pltpu.sync_copy(onesv, slab.at[dstv], add=True)
        return carry
    lax.fori_loop(0, nb, body, 0)
    plsc.subcore_barrier()

    @pl.when(c == 0)
    def _():
        pltpu.sync_copy(slab.at[pl.ds(r0, RPT)], degA.at[pl.ds(r0, RPT)])

    @pl.when(c == 1)
    def _():
        pltpu.sync_copy(slab.at[pl.ds(r0, RPT)], degB.at[pl.ds(r0, RPT)])


def _deg(dst):
    f = pl.kernel(
        _deg_body,
        mesh=plsc.VectorSubcoreMesh(**_MESH),
        out_type=[jax.ShapeDtypeStruct((N, 16), jnp.float32)] * 2,
        scratch_types=[
            pltpu.VMEM((EB, 16), jnp.float32),
            pltpu.VMEM((EB,), jnp.int32),
            pltpu.VMEM_SHARED((N, 16), jnp.float32),
        ],
    )
    ones = jnp.ones((EB, 16), jnp.float32)
    zrows = jnp.zeros((RPT, 16), jnp.float32)
    return f(dst, ones, zrows)


def _c1_body(g0, g1, src_hbm, dst_hbm, zrows, s0, s1, srcv, dstv, rows, slab, sem):
    c = lax.axis_index("c")
    s = lax.axis_index("s")
    r0 = s * RPT
    pltpu.sync_copy(zrows, slab.at[pl.ds(r0, RPT)])
    plsc.subcore_barrier()
    base = pl.multiple_of(s * (E // NS), 8)
    nb = E // (NS * EB)

    @pl.when(c == 0)
    def _():
        _edge_loop(nb, base, src_hbm, dst_hbm, g0, slab, srcv, dstv, rows, sem)

    @pl.when(c == 1)
    def _():
        _edge_loop(nb, base, src_hbm, dst_hbm, g1, slab, srcv, dstv, rows, sem)

    plsc.subcore_barrier()

    @pl.when(c == 0)
    def _():
        pltpu.sync_copy(slab.at[pl.ds(r0, RPT)], s0.at[pl.ds(r0, RPT)])

    @pl.when(c == 1)
    def _():
        pltpu.sync_copy(slab.at[pl.ds(r0, RPT)], s1.at[pl.ds(r0, RPT)])


def _c1(g0, g1, src, dst):
    f = pl.kernel(
        _c1_body,
        mesh=plsc.VectorSubcoreMesh(**_MESH),
        out_type=[jax.ShapeDtypeStruct((N, HALF), jnp.float32)] * 2,
        scratch_types=[
            pltpu.VMEM((EB,), jnp.int32),
            pltpu.VMEM((EB,), jnp.int32),
            pltpu.VMEM((EB, HALF), jnp.float32),
            pltpu.VMEM_SHARED((N, HALF), jnp.float32),
            pltpu.SemaphoreType.DMA,
        ],
    )
    zrows = jnp.zeros((RPT, HALF), jnp.float32)
    return f(g0, g1, src, dst, zrows)


def _c2_body(g2, src_hbm, dst_hbm, zrows, sA, sB, srcv, dstv, rows, slab, sem):
    c = lax.axis_index("c")
    s = lax.axis_index("s")
    r0 = s * RPT
    pltpu.sync_copy(zrows, slab.at[pl.ds(r0, RPT)])
    plsc.subcore_barrier()
    base = pl.multiple_of((c * NS + s) * (E // (NC * NS)), 8)
    nb = E // (NC * NS * EB)
    _edge_loop(nb, base, src_hbm, dst_hbm, g2, slab, srcv, dstv, rows, sem)
    plsc.subcore_barrier()

    @pl.when(c == 0)
    def _():
        pltpu.sync_copy(slab.at[pl.ds(r0, RPT)], sA.at[pl.ds(r0, RPT)])

    @pl.when(c == 1)
    def _():
        pltpu.sync_copy(slab.at[pl.ds(r0, RPT)], sB.at[pl.ds(r0, RPT)])


def _c2(g2, src, dst):
    f = pl.kernel(
        _c2_body,
        mesh=plsc.VectorSubcoreMesh(**_MESH),
        out_type=[jax.ShapeDtypeStruct((N, D_OUT), jnp.float32)] * 2,
        scratch_types=[
            pltpu.VMEM((EB,), jnp.int32),
            pltpu.VMEM((EB,), jnp.int32),
            pltpu.VMEM((EB, D_OUT), jnp.float32),
            pltpu.VMEM_SHARED((N, D_OUT), jnp.float32),
            pltpu.SemaphoreType.DMA,
        ],
    )
    zrows = jnp.zeros((RPT, D_OUT), jnp.float32)
    return f(g2, src, dst, zrows)


BM = 1000


def _b1_body(x_ref, w_ref, da_ref, db_ref, g0_ref, g1_ref, dinv_ref):
    deg = da_ref[:, :1] + db_ref[:, :1] + 1.0
    dinv = lax.rsqrt(deg)
    h = jnp.dot(x_ref[:], w_ref[:], preferred_element_type=jnp.float32)
    g = h * dinv
    g0_ref[:] = g[:, :HALF]
    g1_ref[:] = g[:, HALF:]
    dinv_ref[:] = dinv


def _b1(x, W1, degA, degB):
    return pl.pallas_call(
        _b1_body,
        grid=(N // BM,),
        in_specs=[
            pl.BlockSpec((BM, D_IN), lambda i: (i, 0)),
            pl.BlockSpec((D_IN, D_HID), lambda i: (0, 0)),
            pl.BlockSpec((BM, 16), lambda i: (i, 0)),
            pl.BlockSpec((BM, 16), lambda i: (i, 0)),
        ],
        out_specs=[
            pl.BlockSpec((BM, HALF), lambda i: (i, 0)),
            pl.BlockSpec((BM, HALF), lambda i: (i, 0)),
            pl.BlockSpec((BM, 1), lambda i: (i, 0)),
        ],
        out_shape=[
            jax.ShapeDtypeStruct((N, HALF), jnp.float32),
            jax.ShapeDtypeStruct((N, HALF), jnp.float32),
            jax.ShapeDtypeStruct((N, 1), jnp.float32),
        ],
    )(x, W1, degA, degB)


def _b2_body(s0_ref, s1_ref, g0_ref, g1_ref, dinv_ref, b_ref, w_ref, g2_ref):
    dinv = dinv_ref[:]
    a0 = (s0_ref[:] + g0_ref[:]) * dinv + b_ref[:, :HALF]
    a1 = (s1_ref[:] + g1_ref[:]) * dinv + b_ref[:, HALF:]
    act = jnp.maximum(jnp.concatenate([a0, a1], axis=1), 0.0)
    h2 = jnp.dot(act, w_ref[:], preferred_element_type=jnp.float32)
    g2_ref[:] = h2 * dinv


def _b2(s0, s1, g0, g1, dinv, b1, W2):
    return pl.pallas_call(
        _b2_body,
        grid=(N // BM,),
        in_specs=[
            pl.BlockSpec((BM, HALF), lambda i: (i, 0)),
            pl.BlockSpec((BM, HALF), lambda i: (i, 0)),
            pl.BlockSpec((BM, HALF), lambda i: (i, 0)),
            pl.BlockSpec((BM, HALF), lambda i: (i, 0)),
            pl.BlockSpec((BM, 1), lambda i: (i, 0)),
            pl.BlockSpec((1, D_HID), lambda i: (0, 0)),
            pl.BlockSpec((D_HID, D_OUT), lambda i: (0, 0)),
        ],
        out_specs=pl.BlockSpec((BM, D_OUT), lambda i: (i, 0)),
        out_shape=jax.ShapeDtypeStruct((N, D_OUT), jnp.float32),
    )(s0, s1, g0, g1, dinv, b1, W2)


def _b3_body(sa_ref, sb_ref, g2_ref, dinv_ref, b_ref, out_ref):
    out_ref[:] = (sa_ref[:] + sb_ref[:] + g2_ref[:]) * dinv_ref[:] + b_ref[:]


def _b3(sA, sB, g2, dinv, b2):
    return pl.pallas_call(
        _b3_body,
        grid=(N // BM,),
        in_specs=[
            pl.BlockSpec((BM, D_OUT), lambda i: (i, 0)),
            pl.BlockSpec((BM, D_OUT), lambda i: (i, 0)),
            pl.BlockSpec((BM, D_OUT), lambda i: (i, 0)),
            pl.BlockSpec((BM, 1), lambda i: (i, 0)),
            pl.BlockSpec((1, D_OUT), lambda i: (0, 0)),
        ],
        out_specs=pl.BlockSpec((BM, D_OUT), lambda i: (i, 0)),
        out_shape=jax.ShapeDtypeStruct((N, D_OUT), jnp.float32),
    )(sA, sB, g2, dinv, b2)


def kernel(features, indices, W1, b1, W2, b2):
    src = indices[0]
    dst = indices[1]
    degA, degB = _deg(dst)
    g0, g1, dinv = _b1(features, W1, degA, degB)
    s0, s1 = _c1(g0, g1, src, dst)
    g2 = _b2(s0, s1, g0, g1, dinv, b1.reshape(1, D_HID), W2)
    sA, sB = _c2(g2, src, dst)
    return _b3(sA, sB, g2, dinv, b2.reshape(1, D_OUT))


# speedup vs baseline: 9.9030x; 9.9030x over previous
import jax
import jax.numpy as jnp
from jax import lax
from jax.experimental import pallas as pl
from jax.experimental.pallas import tpu as pltpu
from jax.experimental.pallas import tpu_sc as plsc

N = 10000
E = 320000
D_IN = 128
D_HID = 256
D_OUT = 64
HALF = D_HID // 2

NC = 2
NS = 16
NP = 10112
RPT = NP // NS
EB = 80

_MESH = dict(core_axis_name="c", subcore_axis_name="s")


def _edge_loop(nb, base, src_hbm, dst_hbm, g_hbm, slab, srcv, dstv, rows, sem):
    def body(i, carry):
        off = pl.multiple_of(base + i * EB, 8)
        pltpu.sync_copy(src_hbm.at[pl.ds(off, EB)], srcv)
        pltpu.sync_copy(dst_hbm.at[pl.ds(off, EB)], dstv)
        pltpu.async_copy(g_hbm.at[srcv], rows, sem).wait()
        pltpu.sync_copy(rows, slab.at[dstv], add=True)
        return carry
    lax.fori_loop(0, nb, body, 0)


def _deg_body(dst_hbm, ones_hbm, zrows_hbm, deg2, onesv, dstv, slab):
    c = lax.axis_index("c")
    s = lax.axis_index("s")
    r0 = s * RPT
    pltpu.sync_copy(zrows_hbm, slab.at[pl.ds(r0, RPT)])
    pltpu.sync_copy(ones_hbm, onesv)
    plsc.subcore_barrier()
    base = pl.multiple_of((c * NS + s) * (E // (NC * NS)), 8)
    nb = E // (NC * NS * EB)

    def body(i, carry):
        off = pl.multiple_of(base + i * EB, 8)
        pltpu.sync_copy(dst_hbm.at[pl.ds(off, EB)], dstv)
        pltpu.sync_copy(onesv, slab.at[dstv], add=True)
        return carry
    lax.fori_loop(0, nb, body, 0)
    plsc.subcore_barrier()
    w0 = pl.multiple_of(c * NP + r0, 8)
    pltpu.sync_copy(slab.at[pl.ds(r0, RPT)], deg2.at[pl.ds(w0, RPT)])


def _deg(dst):
    f = pl.kernel(
        _deg_body,
        mesh=plsc.VectorSubcoreMesh(**_MESH),
        out_type=jax.ShapeDtypeStruct((2 * NP, HALF), jnp.float32),
        scratch_types=[
            pltpu.VMEM((EB, HALF), jnp.float32),
            pltpu.VMEM((EB,), jnp.int32),
            pltpu.VMEM_SHARED((NP, HALF), jnp.float32),
        ],
    )
    ones = jnp.ones((EB, HALF), jnp.float32)
    zrows = jnp.zeros((RPT, HALF), jnp.float32)
    deg2 = f(dst, ones, zrows)
    return deg2[:NP], deg2[NP:]


def _c1_body(g0, g1, src_hbm, dst_hbm, zrows, s0, s1, srcv, dstv, rows, slab, sem):
    c = lax.axis_index("c")
    s = lax.axis_index("s")
    r0 = s * RPT
    pltpu.sync_copy(zrows, slab.at[pl.ds(r0, RPT)])
    plsc.subcore_barrier()
    base = pl.multiple_of(s * (E // NS), 8)
    nb = E // (NS * EB)

    @pl.when(c == 0)
    def _():
        _edge_loop(nb, base, src_hbm, dst_hbm, g0, slab, srcv, dstv, rows, sem)

    @pl.when(c == 1)
    def _():
        _edge_loop(nb, base, src_hbm, dst_hbm, g1, slab, srcv, dstv, rows, sem)

    plsc.subcore_barrier()

    @pl.when(c == 0)
    def _():
        pltpu.sync_copy(slab.at[pl.ds(r0, RPT)], s0.at[pl.ds(r0, RPT)])

    @pl.when(c == 1)
    def _():
        pltpu.sync_copy(slab.at[pl.ds(r0, RPT)], s1.at[pl.ds(r0, RPT)])


def _c1(g0, g1, src, dst):
    f = pl.kernel(
        _c1_body,
        mesh=plsc.VectorSubcoreMesh(**_MESH),
        out_type=[jax.ShapeDtypeStruct((NP, HALF), jnp.float32)] * 2,
        scratch_types=[
            pltpu.VMEM((EB,), jnp.int32),
            pltpu.VMEM((EB,), jnp.int32),
            pltpu.VMEM((EB, HALF), jnp.float32),
            pltpu.VMEM_SHARED((NP, HALF), jnp.float32),
            pltpu.SemaphoreType.DMA,
        ],
    )
    zrows = jnp.zeros((RPT, HALF), jnp.float32)
    return f(g0, g1, src, dst, zrows)


def _c2_body(g2, src_hbm, dst_hbm, zrows, sA, sB, srcv, dstv, rows, slab, sem):
    c = lax.axis_index("c")
    s = lax.axis_index("s")
    r0 = s * RPT
    pltpu.sync_copy(zrows, slab.at[pl.ds(r0, RPT)])
    plsc.subcore_barrier()
    base = pl.multiple_of((c * NS + s) * (E // (NC * NS)), 8)
    nb = E // (NC * NS * EB)
    _edge_loop(nb, base, src_hbm, dst_hbm, g2, slab, srcv, dstv, rows, sem)
    plsc.subcore_barrier()

    @pl.when(c == 0)
    def _():
        pltpu.sync_copy(slab.at[pl.ds(r0, RPT)], sA.at[pl.ds(r0, RPT)])

    @pl.when(c == 1)
    def _():
        pltpu.sync_copy(slab.at[pl.ds(r0, RPT)], sB.at[pl.ds(r0, RPT)])


def _c2(g2, src, dst):
    f = pl.kernel(
        _c2_body,
        mesh=plsc.VectorSubcoreMesh(**_MESH),
        out_type=[jax.ShapeDtypeStruct((NP, HALF), jnp.float32)] * 2,
        scratch_types=[
            pltpu.VMEM((EB,), jnp.int32),
            pltpu.VMEM((EB,), jnp.int32),
            pltpu.VMEM((EB, HALF), jnp.float32),
            pltpu.VMEM_SHARED((NP, HALF), jnp.float32),
            pltpu.SemaphoreType.DMA,
        ],
    )
    zrows = jnp.zeros((RPT, HALF), jnp.float32)
    return f(g2, src, dst, zrows)


BM = 1000


def _b1_body(x_ref, w_ref, da_ref, db_ref, g0_ref, g1_ref, dinv_ref):
    deg = da_ref[:, :1] + db_ref[:, :1] + 1.0
    dinv = lax.rsqrt(deg)
    h = jnp.dot(x_ref[:], w_ref[:], preferred_element_type=jnp.float32)
    g = h * dinv
    g0_ref[:] = g[:, :HALF]
    g1_ref[:] = g[:, HALF:]
    dinv_ref[:] = dinv


def _b1(x, W1, degA, degB):
    return pl.pallas_call(
        _b1_body,
        grid=(N // BM,),
        in_specs=[
            pl.BlockSpec((BM, D_IN), lambda i: (i, 0)),
            pl.BlockSpec((D_IN, D_HID), lambda i: (0, 0)),
            pl.BlockSpec((BM, HALF), lambda i: (i, 0)),
            pl.BlockSpec((BM, HALF), lambda i: (i, 0)),
        ],
        out_specs=[
            pl.BlockSpec((BM, HALF), lambda i: (i, 0)),
            pl.BlockSpec((BM, HALF), lambda i: (i, 0)),
            pl.BlockSpec((BM, 1), lambda i: (i, 0)),
        ],
        out_shape=[
            jax.ShapeDtypeStruct((N, HALF), jnp.float32),
            jax.ShapeDtypeStruct((N, HALF), jnp.float32),
            jax.ShapeDtypeStruct((N, 1), jnp.float32),
        ],
    )(x, W1, degA, degB)


def _b2_body(s0_ref, s1_ref, g0_ref, g1_ref, dinv_ref, b_ref, w_ref, g2_ref):
    dinv = dinv_ref[:]
    a0 = (s0_ref[:] + g0_ref[:]) * dinv + b_ref[:, :HALF]
    a1 = (s1_ref[:] + g1_ref[:]) * dinv + b_ref[:, HALF:]
    act = jnp.maximum(jnp.concatenate([a0, a1], axis=1), 0.0)
    h2 = jnp.dot(act, w_ref[:], preferred_element_type=jnp.float32)
    g2_ref[:] = jnp.concatenate(
        [h2 * dinv, jnp.zeros((BM, HALF - D_OUT), jnp.float32)], axis=1)


def _b2(s0, s1, g0, g1, dinv, b1, W2):
    return pl.pallas_call(
        _b2_body,
        grid=(N // BM,),
        in_specs=[
            pl.BlockSpec((BM, HALF), lambda i: (i, 0)),
            pl.BlockSpec((BM, HALF), lambda i: (i, 0)),
            pl.BlockSpec((BM, HALF), lambda i: (i, 0)),
            pl.BlockSpec((BM, HALF), lambda i: (i, 0)),
            pl.BlockSpec((BM, 1), lambda i: (i, 0)),
            pl.BlockSpec((1, D_HID), lambda i: (0, 0)),
            pl.BlockSpec((D_HID, D_OUT), lambda i: (0, 0)),
        ],
        out_specs=pl.BlockSpec((BM, HALF), lambda i: (i, 0)),
        out_shape=jax.ShapeDtypeStruct((N, HALF), jnp.float32),
    )(s0, s1, g0, g1, dinv, b1, W2)


def _b3_body(sa_ref, sb_ref, g2_ref, dinv_ref, b_ref, out_ref):
    tot = sa_ref[:, :D_OUT] + sb_ref[:, :D_OUT] + g2_ref[:, :D_OUT]
    out_ref[:] = tot * dinv_ref[:] + b_ref[:]


def _b3(sA, sB, g2, dinv, b2):
    return pl.pallas_call(
        _b3_body,
        grid=(N // BM,),
        in_specs=[
            pl.BlockSpec((BM, HALF), lambda i: (i, 0)),
            pl.BlockSpec((BM, HALF), lambda i: (i, 0)),
            pl.BlockSpec((BM, HALF), lambda i: (i, 0)),
            pl.BlockSpec((BM, 1), lambda i: (i, 0)),
            pl.BlockSpec((1, D_OUT), lambda i: (0, 0)),
        ],
        out_specs=pl.BlockSpec((BM, D_OUT), lambda i: (i, 0)),
        out_shape=jax.ShapeDtypeStruct((N, D_OUT), jnp.float32),
    )(sA, sB, g2, dinv, b2)


def kernel(features, indices, W1, b1, W2, b2):
    src = indices[0]
    dst = indices[1]
    degA, degB = _deg(dst)
    g0, g1, dinv = _b1(features, W1, degA, degB)
    s0, s1 = _c1(g0, g1, src, dst)
    g2 = _b2(s0, s1, g0, g1, dinv, b1.reshape(1, D_HID), W2)
    sA, sB = _c2(g2, src, dst)
    return _b3(sA, sB, g2, dinv, b2.reshape(1, D_OUT))
